# Initial kernel scaffold; baseline (speedup 1.0000x reference)
import jax
import jax.numpy as jnp
from jax.experimental import pallas as pl


def kernel(x, w1, b1, w2, b2):
    raise NotImplementedError("write your pallas kernel here")


# speedup vs baseline: 1.1417x; 1.1417x over previous
import functools

import jax
import jax.numpy as jnp
from jax.experimental import pallas as pl
from jax.experimental.pallas import tpu as pltpu


def _se_kernel(x_ref, w1_ref, b1_ref, w2_ref, b2_ref, out_ref, *,
               inv_hw, hw):
    n_full = hw // 128
    rem = hw % 128

    if n_full == 0:
        pooled = jnp.sum(x_ref[...], axis=-1, dtype=jnp.float32)
    else:
        acc = x_ref[:, :, 0:128].astype(jnp.float32)
        for j in range(1, n_full):
            acc = acc + x_ref[:, :, j * 128:(j + 1) * 128].astype(jnp.float32)
        pooled = jnp.sum(acc, axis=-1)
        if rem:
            tail = x_ref[:, :, n_full * 128:hw].astype(jnp.float32)
            pooled = pooled + jnp.sum(tail, axis=-1)

    pooled = pooled * inv_hw

    h = jnp.dot(pooled, w1_ref[...], preferred_element_type=jnp.float32)
    h = jnp.maximum(h + b1_ref[...], 0.0)
    y = jnp.dot(h, w2_ref[...], preferred_element_type=jnp.float32)
    out_ref[...] = jax.nn.sigmoid(y + b2_ref[...])


def kernel(x, w1, b1, w2, b2):
    N, C, H, W = x.shape
    hidden = w1.shape[0]
    channel = w2.shape[0]
    HW = H * W
    itemsize = jnp.dtype(x.dtype).itemsize

    budget = 12 * 1024 * 1024
    tn = 1
    for d in range(1, N + 1):
        if N % d == 0 and d * C * HW * itemsize <= budget:
            tn = d
    if N > 1:
        tn = min(tn, max(1, N // 2))
    n_n = N // tn

    x_flat = x.reshape(N, C, HW)
    w1_t = w1.T
    b1_r = b1.reshape(1, hidden)
    w2_t = w2.T
    b2_r = b2.reshape(1, channel)

    kernel_fn = functools.partial(_se_kernel, inv_hw=1.0 / float(HW), hw=HW)

    hw_pad = -(-HW // 128) * 128
    x_block_bytes = tn * C * hw_pad * itemsize
    w_bytes = 4 * (C * hidden + hidden + hidden * channel + channel)
    vmem_limit = int(min(60 * 1024 * 1024,
                         2 * x_block_bytes + 2 * w_bytes
                         + 4 * tn * channel + (4 << 20)))

    cost = pl.CostEstimate(
        flops=int(N * C * HW + 2 * N * C * hidden + 2 * N * hidden * channel),
        transcendentals=int(N * channel),
        bytes_accessed=int(N * C * HW * itemsize + w_bytes + 4 * N * channel),
    )

    out = pl.pallas_call(
        kernel_fn,
        out_shape=jax.ShapeDtypeStruct((N, channel), jnp.float32),
        grid=(n_n,),
        in_specs=[
            pl.BlockSpec((tn, C, HW), lambda n: (n, 0, 0)),
            pl.BlockSpec((C, hidden), lambda n: (0, 0)),
            pl.BlockSpec((1, hidden), lambda n: (0, 0)),
            pl.BlockSpec((hidden, channel), lambda n: (0, 0)),
            pl.BlockSpec((1, channel), lambda n: (0, 0)),
        ],
        out_specs=pl.BlockSpec((tn, channel), lambda n: (n, 0)),
        compiler_params=pltpu.CompilerParams(
            dimension_semantics=("parallel",),
            vmem_limit_bytes=vmem_limit,
        ),
        cost_estimate=cost,
    )(x_flat, w1_t, b1_r, w2_t, b2_r)

    return out.reshape(-1, channel, 1, 1)
